# baseline (speedup 1.0000x reference)
import jax
import jax.numpy as jnp
from jax import lax
from jax.experimental import pallas as pl
from jax.experimental.pallas import tpu as pltpu
from jax.experimental.pallas import tpu_sc as plsc

GRID = 90
R = GRID * GRID
D = 256
L = 16
NC, NS = 2, 16
NW = NC * NS
RPW = 256
NG = RPW // L
HALF = RPW // 2
LAST_START = ((R - RPW) // 8) * 8
TAIL = R - (LAST_START + RPW)
BIG_I = 2 ** 30


def _som_body(x_hbm, w_hbm, dist_out, idx_out, x_v, w_v, bd_v, bi_v,
              sem0, sem1):
    c = lax.axis_index("c")
    s = lax.axis_index("s")
    wid = s * NC + c
    start = jnp.minimum(wid * RPW, LAST_START)
    start = pl.multiple_of(start, 32)

    cp0 = pltpu.make_async_copy(
        w_hbm.at[pl.ds(start, HALF)], w_v.at[pl.ds(0, HALF)], sem0)
    cp0.start()
    cp1 = pltpu.make_async_copy(
        w_hbm.at[pl.ds(start + HALF, HALF)], w_v.at[pl.ds(HALF, HALF)], sem1)
    cp1.start()
    pltpu.sync_copy(x_hbm, x_v)

    @pl.when(wid == NW - 1)
    def _():
        pltpu.sync_copy(w_hbm.at[pl.ds(LAST_START + RPW, TAIL)],
                        w_v.at[pl.ds(RPW, TAIL)])

    lane = lax.iota(jnp.int32, L)
    rows = [lane + g * L for g in range(NG + 1)]
    last = wid == NW - 1

    def make_pass(groups):
        def feat_step(j, accs):
            cols = (lane + j) & (D - 1)
            xj = plsc.load_gather(x_v, [cols])
            out = []
            for k, g in enumerate(groups):
                wv = plsc.load_gather(w_v, [rows[g], cols])
                dv = wv - xj
                out.append(accs[k] + dv * dv)
            return tuple(out)
        return feat_step

    zero = jnp.zeros((L,), jnp.float32)
    g1 = list(range(NG // 2))
    g2 = list(range(NG // 2, NG + 1))
    cp0.wait()
    accs1 = lax.fori_loop(0, D, make_pass(g1), (zero,) * len(g1))
    cp1.wait()
    accs2 = lax.fori_loop(0, D, make_pass(g2), (zero,) * len(g2))

    best_d = jnp.full((L,), jnp.inf, jnp.float32)
    best_i = jnp.full((L,), BIG_I, jnp.int32)
    for g, d in zip(g1 + g2, list(accs1) + list(accs2)):
        gi = start.astype(jnp.int32) + rows[g]
        better = ((d < best_d) | ((d == best_d) & (gi < best_i))) & (gi < R)
        if g == NG:
            better = better & last
        best_d = jnp.where(better, d, best_d)
        best_i = jnp.where(better, gi, best_i)

    bd_v[...] = best_d
    bi_v[...] = best_i
    pltpu.sync_copy(bd_v, dist_out.at[wid])
    pltpu.sync_copy(bi_v, idx_out.at[wid])


def _som_call(inputs, w):
    return pl.kernel(
        _som_body,
        mesh=plsc.VectorSubcoreMesh(core_axis_name="c", subcore_axis_name="s"),
        out_type=[
            jax.ShapeDtypeStruct((NW, L), jnp.float32),
            jax.ShapeDtypeStruct((NW, L), jnp.int32),
        ],
        scratch_types=[
            pltpu.VMEM((D,), jnp.float32),
            pltpu.VMEM((RPW + L, D), jnp.float32),
            pltpu.VMEM((L,), jnp.float32),
            pltpu.VMEM((L,), jnp.int32),
            pltpu.SemaphoreType.DMA,
            pltpu.SemaphoreType.DMA,
        ],
        compiler_params=pltpu.CompilerParams(needs_layout_passes=False),
    )(inputs, w)


def _merge_body(d_ref, i_ref, o_ref):
    d = d_ref[...]
    i = i_ref[...]
    m = jnp.min(d)
    best = jnp.min(jnp.where(d == m, i, BIG_I))
    o_ref[0] = best // GRID
    o_ref[1] = best - (best // GRID) * GRID


def kernel(inputs, w):
    dists, idxs = _som_call(inputs, w)
    out = pl.pallas_call(
        _merge_body,
        out_shape=jax.ShapeDtypeStruct((2,), jnp.int32),
        out_specs=pl.BlockSpec(memory_space=pltpu.SMEM),
    )(dists, idxs)
    return out.astype(jnp.int64)
